# baseline (speedup 1.0000x reference)
import functools

import jax
import jax.numpy as jnp
from jax import lax
from jax.experimental import pallas as pl
from jax.experimental.pallas import tpu as pltpu
from jax.experimental.pallas import tpu_sc as plsc

B = 16384
C = 1000
NC = 2
NS = 16
NW = NC * NS
RPW = B // NW
ZBLK = 512
TBL = 16384


def _zero_body(t_ref, o_ref):
    o_ref[...] = jnp.zeros_like(o_ref)


_zero_fill = pl.pallas_call(
    _zero_body,
    out_shape=jax.ShapeDtypeStruct((B, C), jnp.float32),
    grid=(B // ZBLK,),
    in_specs=[pl.BlockSpec(memory_space=pl.ANY)],
    out_specs=pl.BlockSpec((ZBLK, C), lambda i: (i, 0)),
)


def _ones_body(tgt_hbm, out_hbm, idx_v, table_v, drain_v, sem):
    cid = lax.axis_index("c")
    sid = lax.axis_index("s")
    wid = sid * NC + cid
    base = pl.multiple_of(wid * RPW, 8)

    lanes = lax.iota(jnp.int32, 16)
    for r in range(8):
        table_v[pl.ds(1024 + 2048 * r, 16)] = \
            jnp.where(lanes == r, 1.0, 0.0)

    pltpu.sync_copy(tgt_hbm.at[pl.ds(base, RPW)], idx_v)

    @pl.loop(0, RPW // 16)
    def _grp(g):
        c16 = idx_v[pl.ds(pl.multiple_of(g * 16, 16), 16)]
        o16 = 1024 + jnp.bitwise_and(c16, 7) * 2048
        cb16 = jnp.bitwise_and(c16, ~7)
        for k in range(16):
            o = pl.multiple_of(o16[k], 8)
            cb = pl.multiple_of(cb16[k], 8)
            row = base + g * 16 + k
            pltpu.async_copy(table_v.at[pl.ds(o, 8)],
                             out_hbm.at[row, pl.ds(cb, 8)], sem)

    pltpu.make_async_copy(tgt_hbm.at[pl.ds(0, 4096)], drain_v, sem).wait()


_sc_ones = pl.kernel(
    _ones_body,
    out_type=(),
    mesh=plsc.VectorSubcoreMesh(core_axis_name="c", subcore_axis_name="s"),
    compiler_params=pltpu.CompilerParams(needs_layout_passes=False),
    scratch_types=[
        pltpu.VMEM((RPW,), jnp.int32),
        pltpu.VMEM((TBL,), jnp.float32),
        pltpu.VMEM((4096,), jnp.int32),
        pltpu.SemaphoreType.DMA,
    ],
)


def kernel(target):
    tgt = target.astype(jnp.int32)
    out = jax.new_ref(_zero_fill(tgt))
    _sc_ones(tgt, out)
    return out[...]


# speedup vs baseline: 1.0053x; 1.0053x over previous
import jax
import jax.numpy as jnp
from jax import lax
from jax.experimental import pallas as pl
from jax.experimental.pallas import tpu as pltpu
from jax.experimental.pallas import tpu_sc as plsc

B = 16384
C = 1000
NC = 2
NS = 16
NW = NC * NS
RPW = B // NW
ZBLK = 512
TBL = 16384


def _zero_body(t_ref, o_ref):
    o_ref[...] = jnp.zeros_like(o_ref)


_zero_fill = pl.pallas_call(
    _zero_body,
    out_shape=jax.ShapeDtypeStruct((B, C), jnp.float32),
    grid=(B // ZBLK,),
    in_specs=[pl.BlockSpec(memory_space=pl.ANY)],
    out_specs=pl.BlockSpec((ZBLK, C), lambda i: (i, 0)),
)


def _ones_body(tgt_hbm, out_hbm, idx_v, table_v, drain_v, sem):
    cid = lax.axis_index("c")
    sid = lax.axis_index("s")
    wid = sid * NC + cid
    base = pl.multiple_of(wid * RPW, 8)

    lanes = lax.iota(jnp.int32, 16)
    for r in range(8):
        table_v[pl.ds(1024 + 2048 * r, 16)] = \
            jnp.where(lanes == r, 1.0, 0.0)

    pltpu.sync_copy(tgt_hbm.at[pl.ds(base, RPW)], idx_v)

    @pl.loop(0, RPW // 16)
    def _grp(g):
        c16 = idx_v[pl.ds(pl.multiple_of(g * 16, 16), 16)]
        o16 = 1024 + jnp.bitwise_and(c16, 7) * 2048
        cb16 = jnp.bitwise_and(c16, ~7)
        for k in range(16):
            o = pl.multiple_of(o16[k], 8)
            cb = pl.multiple_of(cb16[k], 8)
            row = base + g * 16 + k
            pltpu.async_copy(table_v.at[pl.ds(o, 8)],
                             out_hbm.at[row, pl.ds(cb, 8)], sem)

    pltpu.make_async_copy(tgt_hbm.at[pl.ds(0, 4096)], drain_v, sem).wait()


_sc_ones = pl.kernel(
    _ones_body,
    out_type=(),
    mesh=plsc.VectorSubcoreMesh(core_axis_name="c", subcore_axis_name="s"),
    compiler_params=pltpu.CompilerParams(needs_layout_passes=False),
    scratch_types=[
        pltpu.VMEM((RPW,), jnp.int32),
        pltpu.VMEM((TBL,), jnp.float32),
        pltpu.VMEM((4096,), jnp.int32),
        pltpu.SemaphoreType.DMA,
    ],
)


def kernel(target):
    tgt = target.astype(jnp.int32)
    out = jax.new_ref(_zero_fill(tgt))
    _sc_ones(tgt, out)
    return jax.freeze(out)


# speedup vs baseline: 1.0282x; 1.0228x over previous
import jax
import jax.numpy as jnp
from jax import lax
from jax.experimental import pallas as pl
from jax.experimental.pallas import tpu as pltpu
from jax.experimental.pallas import tpu_sc as plsc

B = 16384
C = 1000
NC = 2
NS = 16
NW = NC * NS
RPW = B // NW
ZBLK = 512
TBL = 16384


def _zero_body(out_hbm, zb, sem):
    zb[...] = jnp.zeros_like(zb)
    for i in range(B // ZBLK):
        pltpu.async_copy(zb, out_hbm.at[pl.ds(i * ZBLK, ZBLK)], sem)
    pltpu.make_async_copy(out_hbm, out_hbm, sem).wait()


_tc_mesh = pltpu.create_tensorcore_mesh("tc")

_zero_fill = pl.kernel(
    _zero_body,
    out_type=(),
    mesh=_tc_mesh,
    scratch_types=[
        pltpu.VMEM((ZBLK, C), jnp.float32),
        pltpu.SemaphoreType.DMA,
    ],
)


def _ones_body(tgt_hbm, out_hbm, idx_v, table_v, drain_v, sem):
    cid = lax.axis_index("c")
    sid = lax.axis_index("s")
    wid = sid * NC + cid
    base = pl.multiple_of(wid * RPW, 8)

    lanes = lax.iota(jnp.int32, 16)
    for r in range(8):
        table_v[pl.ds(1024 + 2048 * r, 16)] = \
            jnp.where(lanes == r, 1.0, 0.0)

    pltpu.sync_copy(tgt_hbm.at[pl.ds(base, RPW)], idx_v)

    @pl.loop(0, RPW // 16)
    def _grp(g):
        c16 = idx_v[pl.ds(pl.multiple_of(g * 16, 16), 16)]
        o16 = 1024 + jnp.bitwise_and(c16, 7) * 2048
        cb16 = jnp.bitwise_and(c16, ~7)
        for k in range(16):
            o = pl.multiple_of(o16[k], 8)
            cb = pl.multiple_of(cb16[k], 8)
            row = base + g * 16 + k
            pltpu.async_copy(table_v.at[pl.ds(o, 8)],
                             out_hbm.at[row, pl.ds(cb, 8)], sem)

    pltpu.make_async_copy(tgt_hbm.at[pl.ds(0, 4096)], drain_v, sem).wait()


_sc_ones = pl.kernel(
    _ones_body,
    out_type=(),
    mesh=plsc.VectorSubcoreMesh(core_axis_name="c", subcore_axis_name="s"),
    compiler_params=pltpu.CompilerParams(needs_layout_passes=False),
    scratch_types=[
        pltpu.VMEM((RPW,), jnp.int32),
        pltpu.VMEM((TBL,), jnp.float32),
        pltpu.VMEM((4096,), jnp.int32),
        pltpu.SemaphoreType.DMA,
    ],
)


def kernel(target):
    tgt = target.astype(jnp.int32)
    out = jax.ref.empty_ref(jax.ShapeDtypeStruct((B, C), jnp.float32))
    _zero_fill(out)
    _sc_ones(tgt, out)
    return jax.freeze(out)
